# baseline (speedup 1.0000x reference)
import functools

import jax
import jax.numpy as jnp
from jax import lax
from jax.experimental import pallas as pl
from jax.experimental.pallas import tpu as pltpu
from jax.experimental.pallas import tpu_sc as plsc

N = 10000
E = 320000
D_IN = 128
H = 256
HEADS = 4
DH = 64
NOTES = 128
L = 3

ROW_BLK = 1000


def _ln(z):
    m = jnp.mean(z, axis=-1, keepdims=True)
    v = jnp.mean((z - m) ** 2, axis=-1, keepdims=True)
    return (z - m) * lax.rsqrt(v + 1e-5)


def _mlp_block_kernel(hn_ref, agg_ref, res_ref, eps_ref, w0_ref, b0_ref,
                      w1_ref, b1_ref, w2_ref, b2_ref, h_out_ref, hn_out_ref,
                      *, has_residual):
    z = (1.0 + eps_ref[0, 0]) * hn_ref[...] + agg_ref[...]
    z = jnp.maximum(
        jnp.dot(z, w0_ref[...], preferred_element_type=jnp.float32)
        + b0_ref[...], 0.0)
    z = jnp.maximum(
        jnp.dot(z, w1_ref[...], preferred_element_type=jnp.float32)
        + b1_ref[...], 0.0)
    z = jnp.dot(z, w2_ref[...], preferred_element_type=jnp.float32) + b2_ref[...]
    z = _ln(z)
    if has_residual:
        z = z + res_ref[...]
    h_out_ref[...] = z
    hn_out_ref[...] = _ln(z)


def _mlp_block(hn, agg, res, eps, w0, b0, w1, b1, w2, b2):
    n, in_d = hn.shape
    has_residual = res is not None
    grid = (n // ROW_BLK,)
    in_specs = [
        pl.BlockSpec((ROW_BLK, in_d), lambda i: (i, 0)),
        pl.BlockSpec((ROW_BLK, in_d), lambda i: (i, 0)),
    ]
    args = [hn, agg]
    if has_residual:
        in_specs.append(pl.BlockSpec((ROW_BLK, H), lambda i: (i, 0)))
        args.append(res)
    else:
        in_specs.append(pl.BlockSpec(memory_space=pltpu.SMEM))
        args.append(jnp.zeros((1,), jnp.float32))
    in_specs.append(pl.BlockSpec(memory_space=pltpu.SMEM))
    args.append(eps.reshape(1, 1))
    for w, b in ((w0, b0), (w1, b1), (w2, b2)):
        in_specs.append(pl.BlockSpec((w.shape[0], H), lambda i: (0, 0)))
        in_specs.append(pl.BlockSpec((1, H), lambda i: (0, 0)))
        args.extend([w, b.reshape(1, H)])
    out_specs = [
        pl.BlockSpec((ROW_BLK, H), lambda i: (i, 0)),
        pl.BlockSpec((ROW_BLK, H), lambda i: (i, 0)),
    ]
    return pl.pallas_call(
        functools.partial(_mlp_block_kernel, has_residual=has_residual),
        grid=grid,
        in_specs=in_specs,
        out_specs=out_specs,
        out_shape=[
            jax.ShapeDtypeStruct((n, H), jnp.float32),
            jax.ShapeDtypeStruct((n, H), jnp.float32),
        ],
    )(*args)


def _ln_kernel(x_ref, o_ref):
    o_ref[...] = _ln(x_ref[...])


def _ln_call(x):
    n, d = x.shape
    return pl.pallas_call(
        _ln_kernel,
        grid=(n // ROW_BLK,),
        in_specs=[pl.BlockSpec((ROW_BLK, d), lambda i: (i, 0))],
        out_specs=pl.BlockSpec((ROW_BLK, d), lambda i: (i, 0)),
        out_shape=jax.ShapeDtypeStruct((n, d), jnp.float32),
    )(x)


NC, NS = 2, 16
CB = 800
NCHUNK = E // CB
NBUF = 4
GSUB = (0, 128, 256, 384, 512, 640, 768)
GLEN = (128, 128, 128, 128, 128, 128, 32)


@functools.lru_cache(maxsize=None)
def _make_segsum(d):
    k_blocks = d // 8
    shift = k_blocks.bit_length() - 1
    mesh = plsc.VectorSubcoreMesh(core_axis_name="c", subcore_axis_name="s")

    @functools.partial(
        pl.kernel,
        out_type=jax.ShapeDtypeStruct((d // 8, N, 8), jnp.float32),
        mesh=mesh,
        scratch_types=[
            [pltpu.VMEM((CB,), jnp.int32)] * NBUF,
            [pltpu.VMEM((CB,), jnp.int32)] * NBUF,
            [pltpu.VMEM((CB,), jnp.int32)] * NBUF,
            [pltpu.VMEM((CB, 8), jnp.float32)] * NBUF,
            pltpu.VMEM((N, 8), jnp.float32),
            [pltpu.SemaphoreType.DMA] * NBUF,
            [pltpu.SemaphoreType.DMA] * NBUF,
        ],
        compiler_params=pltpu.CompilerParams(use_tc_tiling_on_sc=False,
                                             needs_layout_passes=False),
    )
    def segsum(hn_hbm, src_hbm, dst_hbm, zeros_hbm, out_hbm,
               sbuf, dbuf, gidx, rows, acc, isem, gsem):
        c = lax.axis_index("c")
        s = lax.axis_index("s")
        w = s * NC + c

        pair_pat = jnp.where(lax.iota(jnp.int32, 16) < 8, 0, 1)
        col_pat = lax.iota(jnp.int32, 16) & 7
        m_lo = lax.iota(jnp.int32, 16) < 8
        m_hi = jnp.logical_not(m_lo)

        def fire_idx(i, b):
            off = i * CB
            pltpu.async_copy(src_hbm.at[pl.ds(off, CB)], sbuf[b], isem[b])
            pltpu.async_copy(dst_hbm.at[pl.ds(off, CB)], dbuf[b], isem[b])

        def wait_idx(b):
            pltpu.make_async_copy(src_hbm.at[pl.ds(0, CB)], sbuf[b],
                                  isem[b]).wait()
            pltpu.make_async_copy(dst_hbm.at[pl.ds(0, CB)], dbuf[b],
                                  isem[b]).wait()

        def compute_gidx(b, blk):
            def idx_body(q, carry):
                sv = sbuf[b][pl.ds(q * 16, 16)]
                gidx[b][pl.ds(q * 16, 16)] = (sv << shift) + blk
                return carry

            lax.fori_loop(0, CB // 16, idx_body, 0)

        def fire_gather(b):
            for o, n in zip(GSUB, GLEN):
                pltpu.async_copy(hn_hbm.at[gidx[b].at[pl.ds(o, n)]],
                                 rows[b].at[pl.ds(o, n)], gsem[b])

        def wait_gather(b):
            for o, n in zip(GSUB, GLEN):
                pltpu.make_async_copy(hn_hbm.at[gidx[b].at[pl.ds(o, n)]],
                                      rows[b].at[pl.ds(o, n)],
                                      gsem[b]).wait()

        def add_chunk(b):
            def add_body(g, ridx):
                for p in range(16):
                    dpair = plsc.load_gather(dbuf[b], [ridx])
                    v = plsc.load_gather(rows[b], [ridx, col_pat])
                    plsc.addupdate_scatter(acc, [dpair, col_pat], v,
                                           mask=m_lo)
                    plsc.addupdate_scatter(acc, [dpair, col_pat], v,
                                           mask=m_hi)
                    ridx = ridx + 2
                return ridx

            lax.fori_loop(0, CB // 32, add_body, pair_pat)

        @pl.when(w < k_blocks)
        def _():
            blk = w
            pltpu.sync_copy(zeros_hbm, acc)

            fire_idx(0, 0)
            fire_idx(1, 1)
            wait_idx(0)
            compute_gidx(0, blk)
            fire_gather(0)

            def super_body(j, carry):
                for b in range(NBUF):
                    i = j * NBUF + b
                    bn = (b + 1) % NBUF
                    bnn = (b + 2) % NBUF

                    @pl.when(i + 1 < NCHUNK)
                    def _():
                        wait_idx(bn)
                        compute_gidx(bn, blk)
                        fire_gather(bn)

                    @pl.when(i + 2 < NCHUNK)
                    def _():
                        fire_idx(i + 2, bnn)

                    wait_gather(b)
                    add_chunk(b)
                return carry

            lax.fori_loop(0, NCHUNK // NBUF, super_body, 0)
            pltpu.sync_copy(acc, out_hbm.at[blk])

    return segsum


def _readout_kernel(h_ref, wk_ref, wv_ref, s_ref, wo_ref, wn_ref, bn_ref,
                    embed_ref, logits_ref):
    h = h_ref[...]
    k = jnp.dot(h, wk_ref[...], preferred_element_type=jnp.float32)
    s = jnp.dot(k, s_ref[...], preferred_element_type=jnp.float32)
    m = jnp.max(s, axis=0, keepdims=True)
    w = jnp.exp(s - m)
    l = jnp.sum(w, axis=0, keepdims=True)
    attn = w / l
    v = jnp.dot(h, wv_ref[...], preferred_element_type=jnp.float32)
    wexp = jnp.repeat(attn, DH, axis=1)
    pooled = jnp.sum(wexp * v, axis=0, keepdims=True)
    embed = jnp.dot(pooled, wo_ref[...], preferred_element_type=jnp.float32)
    embed_ref[...] = embed
    logits_ref[...] = (
        jnp.dot(embed, wn_ref[...], preferred_element_type=jnp.float32)
        + bn_ref[...])


def _readout(h, wk, wv, seed, wo, wn, bn):
    smat = (jnp.eye(HEADS, dtype=jnp.float32)[:, None, :]
            * seed[:, :, None]).reshape(H, HEADS) / jnp.sqrt(float(DH))
    return pl.pallas_call(
        _readout_kernel,
        in_specs=[pl.BlockSpec((N, H), lambda: (0, 0)),
                  pl.BlockSpec((H, H), lambda: (0, 0)),
                  pl.BlockSpec((H, H), lambda: (0, 0)),
                  pl.BlockSpec((H, HEADS), lambda: (0, 0)),
                  pl.BlockSpec((H, H), lambda: (0, 0)),
                  pl.BlockSpec((H, NOTES), lambda: (0, 0)),
                  pl.BlockSpec((1, NOTES), lambda: (0, 0))],
        out_specs=[pl.BlockSpec((1, H), lambda: (0, 0)),
                   pl.BlockSpec((1, NOTES), lambda: (0, 0))],
        out_shape=[jax.ShapeDtypeStruct((1, H), jnp.float32),
                   jax.ShapeDtypeStruct((1, NOTES), jnp.float32)],
    )(h, wk, wv, smat, wo, wn, bn.reshape(1, NOTES))


def kernel(x, params, edge_index):
    src = edge_index[0]
    dst = edge_index[1]
    zeros_acc = jnp.zeros((N, 8), jnp.float32)

    hn = _ln_call(x)
    h = None
    residual = None
    for l in range(L):
        d = hn.shape[1]
        hn_r = hn.reshape(N * (d // 8), 8)
        agg_t = _make_segsum(d)(hn_r, src, dst, zeros_acc)
        agg = agg_t.transpose(1, 0, 2).reshape(N, d)
        h, hn = _mlp_block(
            hn, agg, residual, params["eps%d" % l],
            params["W%d_0" % l], params["b%d_0" % l],
            params["W%d_1" % l], params["b%d_1" % l],
            params["W%d_2" % l], params["b%d_2" % l])
        residual = h

    embed, logits = _readout(h, params["Wk"], params["Wv"], params["seed"],
                             params["Wo"], params["Wn"], params["bn"])
    return (embed, logits)


# speedup vs baseline: 1.0007x; 1.0007x over previous
import functools

import jax
import jax.numpy as jnp
from jax import lax
from jax.experimental import pallas as pl
from jax.experimental.pallas import tpu as pltpu
from jax.experimental.pallas import tpu_sc as plsc

N = 10000
E = 320000
D_IN = 128
H = 256
HEADS = 4
DH = 64
NOTES = 128
L = 3

ROW_BLK = 1000


def _ln(z):
    m = jnp.mean(z, axis=-1, keepdims=True)
    v = jnp.mean((z - m) ** 2, axis=-1, keepdims=True)
    return (z - m) * lax.rsqrt(v + 1e-5)


def _mlp_block_kernel(hn_ref, agg_ref, res_ref, eps_ref, w0_ref, b0_ref,
                      w1_ref, b1_ref, w2_ref, b2_ref, h_out_ref, hn_out_ref,
                      *, has_residual):
    z = (1.0 + eps_ref[0, 0]) * hn_ref[...] + agg_ref[...]
    z = jnp.maximum(
        jnp.dot(z, w0_ref[...], preferred_element_type=jnp.float32)
        + b0_ref[...], 0.0)
    z = jnp.maximum(
        jnp.dot(z, w1_ref[...], preferred_element_type=jnp.float32)
        + b1_ref[...], 0.0)
    z = jnp.dot(z, w2_ref[...], preferred_element_type=jnp.float32) + b2_ref[...]
    z = _ln(z)
    if has_residual:
        z = z + res_ref[...]
    h_out_ref[...] = z
    hn_out_ref[...] = _ln(z)


def _mlp_block(hn, agg, res, eps, w0, b0, w1, b1, w2, b2):
    n, in_d = hn.shape
    has_residual = res is not None
    grid = (n // ROW_BLK,)
    in_specs = [
        pl.BlockSpec((ROW_BLK, in_d), lambda i: (i, 0)),
        pl.BlockSpec((ROW_BLK, in_d), lambda i: (i, 0)),
    ]
    args = [hn, agg]
    if has_residual:
        in_specs.append(pl.BlockSpec((ROW_BLK, H), lambda i: (i, 0)))
        args.append(res)
    else:
        in_specs.append(pl.BlockSpec(memory_space=pltpu.SMEM))
        args.append(jnp.zeros((1,), jnp.float32))
    in_specs.append(pl.BlockSpec(memory_space=pltpu.SMEM))
    args.append(eps.reshape(1, 1))
    for w, b in ((w0, b0), (w1, b1), (w2, b2)):
        in_specs.append(pl.BlockSpec((w.shape[0], H), lambda i: (0, 0)))
        in_specs.append(pl.BlockSpec((1, H), lambda i: (0, 0)))
        args.extend([w, b.reshape(1, H)])
    out_specs = [
        pl.BlockSpec((ROW_BLK, H), lambda i: (i, 0)),
        pl.BlockSpec((ROW_BLK, H), lambda i: (i, 0)),
    ]
    return pl.pallas_call(
        functools.partial(_mlp_block_kernel, has_residual=has_residual),
        grid=grid,
        in_specs=in_specs,
        out_specs=out_specs,
        out_shape=[
            jax.ShapeDtypeStruct((n, H), jnp.float32),
            jax.ShapeDtypeStruct((n, H), jnp.float32),
        ],
    )(*args)


def _ln_kernel(x_ref, o_ref):
    o_ref[...] = _ln(x_ref[...])


def _ln_call(x):
    n, d = x.shape
    return pl.pallas_call(
        _ln_kernel,
        grid=(n // ROW_BLK,),
        in_specs=[pl.BlockSpec((ROW_BLK, d), lambda i: (i, 0))],
        out_specs=pl.BlockSpec((ROW_BLK, d), lambda i: (i, 0)),
        out_shape=jax.ShapeDtypeStruct((n, d), jnp.float32),
    )(x)


NC, NS = 2, 16
CB = 1280
NCHUNK = E // CB
NBUF = 3
GSUB = tuple(range(0, CB, 128))
GLEN = (128,) * (CB // 128)


@functools.lru_cache(maxsize=None)
def _make_segsum(d):
    k_blocks = d // 8
    shift = k_blocks.bit_length() - 1
    mesh = plsc.VectorSubcoreMesh(core_axis_name="c", subcore_axis_name="s")

    @functools.partial(
        pl.kernel,
        out_type=jax.ShapeDtypeStruct((d // 8, N, 8), jnp.float32),
        mesh=mesh,
        scratch_types=[
            [pltpu.VMEM((CB,), jnp.int32)] * NBUF,
            [pltpu.VMEM((CB,), jnp.int32)] * NBUF,
            [pltpu.VMEM((CB,), jnp.int32)] * NBUF,
            [pltpu.VMEM((CB, 8), jnp.float32)] * NBUF,
            pltpu.VMEM((N, 8), jnp.float32),
            [pltpu.SemaphoreType.DMA] * NBUF,
            [pltpu.SemaphoreType.DMA] * NBUF,
        ],
        compiler_params=pltpu.CompilerParams(use_tc_tiling_on_sc=False,
                                             needs_layout_passes=False),
    )
    def segsum(hn_hbm, src_hbm, dst_hbm, zeros_hbm, out_hbm,
               sbuf, dbuf, gidx, rows, acc, isem, gsem):
        c = lax.axis_index("c")
        s = lax.axis_index("s")
        w = s * NC + c

        pair_pat = jnp.where(lax.iota(jnp.int32, 16) < 8, 0, 1)
        col_pat = lax.iota(jnp.int32, 16) & 7
        m_lo = lax.iota(jnp.int32, 16) < 8
        m_hi = jnp.logical_not(m_lo)

        def fire_idx(i, b):
            off = i * CB
            pltpu.async_copy(src_hbm.at[pl.ds(off, CB)], sbuf[b], isem[b])
            pltpu.async_copy(dst_hbm.at[pl.ds(off, CB)], dbuf[b], isem[b])

        def wait_idx(b):
            pltpu.make_async_copy(src_hbm.at[pl.ds(0, CB)], sbuf[b],
                                  isem[b]).wait()
            pltpu.make_async_copy(dst_hbm.at[pl.ds(0, CB)], dbuf[b],
                                  isem[b]).wait()

        def compute_gidx(b, blk):
            def idx_body(q, carry):
                sv = sbuf[b][pl.ds(q * 16, 16)]
                gidx[b][pl.ds(q * 16, 16)] = (sv << shift) + blk
                return carry

            lax.fori_loop(0, CB // 16, idx_body, 0)

        def fire_gather(b):
            for o, n in zip(GSUB, GLEN):
                pltpu.async_copy(hn_hbm.at[gidx[b].at[pl.ds(o, n)]],
                                 rows[b].at[pl.ds(o, n)], gsem[b])

        def wait_gather(b):
            for o, n in zip(GSUB, GLEN):
                pltpu.make_async_copy(hn_hbm.at[gidx[b].at[pl.ds(o, n)]],
                                      rows[b].at[pl.ds(o, n)],
                                      gsem[b]).wait()

        def add_chunk(b):
            def add_body(g, ridx):
                for p in range(16):
                    dpair = plsc.load_gather(dbuf[b], [ridx])
                    v = plsc.load_gather(rows[b], [ridx, col_pat])
                    plsc.addupdate_scatter(acc, [dpair, col_pat], v,
                                           mask=m_lo)
                    plsc.addupdate_scatter(acc, [dpair, col_pat], v,
                                           mask=m_hi)
                    ridx = ridx + 2
                return ridx

            lax.fori_loop(0, CB // 32, add_body, pair_pat)

        @pl.when(w < k_blocks)
        def _():
            blk = w
            pltpu.sync_copy(zeros_hbm, acc)

            fire_idx(0, 0)
            fire_idx(1, 1)
            wait_idx(0)
            compute_gidx(0, blk)
            fire_gather(0)

            def super_body(j, carry):
                for b in range(NBUF):
                    i = j * NBUF + b
                    bn = (b + 1) % NBUF
                    bnn = (b + 2) % NBUF

                    @pl.when(i + 1 < NCHUNK)
                    def _():
                        wait_idx(bn)
                        compute_gidx(bn, blk)
                        fire_gather(bn)

                    @pl.when(i + 2 < NCHUNK)
                    def _():
                        fire_idx(i + 2, bnn)

                    wait_gather(b)
                    add_chunk(b)
                return carry

            lax.fori_loop(0, NCHUNK // NBUF, super_body, 0)
            for r in range(NCHUNK - (NCHUNK // NBUF) * NBUF):
                b = ((NCHUNK // NBUF) * NBUF + r) % NBUF
                wait_gather(b)
                add_chunk(b)
            pltpu.sync_copy(acc, out_hbm.at[blk])

    return segsum


def _readout_kernel(h_ref, wk_ref, wv_ref, s_ref, wo_ref, wn_ref, bn_ref,
                    embed_ref, logits_ref):
    h = h_ref[...]
    k = jnp.dot(h, wk_ref[...], preferred_element_type=jnp.float32)
    s = jnp.dot(k, s_ref[...], preferred_element_type=jnp.float32)
    m = jnp.max(s, axis=0, keepdims=True)
    w = jnp.exp(s - m)
    l = jnp.sum(w, axis=0, keepdims=True)
    attn = w / l
    v = jnp.dot(h, wv_ref[...], preferred_element_type=jnp.float32)
    wexp = jnp.repeat(attn, DH, axis=1)
    pooled = jnp.sum(wexp * v, axis=0, keepdims=True)
    embed = jnp.dot(pooled, wo_ref[...], preferred_element_type=jnp.float32)
    embed_ref[...] = embed
    logits_ref[...] = (
        jnp.dot(embed, wn_ref[...], preferred_element_type=jnp.float32)
        + bn_ref[...])


def _readout(h, wk, wv, seed, wo, wn, bn):
    smat = (jnp.eye(HEADS, dtype=jnp.float32)[:, None, :]
            * seed[:, :, None]).reshape(H, HEADS) / jnp.sqrt(float(DH))
    return pl.pallas_call(
        _readout_kernel,
        in_specs=[pl.BlockSpec((N, H), lambda: (0, 0)),
                  pl.BlockSpec((H, H), lambda: (0, 0)),
                  pl.BlockSpec((H, H), lambda: (0, 0)),
                  pl.BlockSpec((H, HEADS), lambda: (0, 0)),
                  pl.BlockSpec((H, H), lambda: (0, 0)),
                  pl.BlockSpec((H, NOTES), lambda: (0, 0)),
                  pl.BlockSpec((1, NOTES), lambda: (0, 0))],
        out_specs=[pl.BlockSpec((1, H), lambda: (0, 0)),
                   pl.BlockSpec((1, NOTES), lambda: (0, 0))],
        out_shape=[jax.ShapeDtypeStruct((1, H), jnp.float32),
                   jax.ShapeDtypeStruct((1, NOTES), jnp.float32)],
    )(h, wk, wv, smat, wo, wn, bn.reshape(1, NOTES))


def kernel(x, params, edge_index):
    src = edge_index[0]
    dst = edge_index[1]
    zeros_acc = jnp.zeros((N, 8), jnp.float32)

    hn = _ln_call(x)
    h = None
    residual = None
    for l in range(L):
        d = hn.shape[1]
        hn_r = hn.reshape(N * (d // 8), 8)
        agg_t = _make_segsum(d)(hn_r, src, dst, zeros_acc)
        agg = agg_t.transpose(1, 0, 2).reshape(N, d)
        h, hn = _mlp_block(
            hn, agg, residual, params["eps%d" % l],
            params["W%d_0" % l], params["b%d_0" % l],
            params["W%d_1" % l], params["b%d_1" % l],
            params["W%d_2" % l], params["b%d_2" % l])
        residual = h

    embed, logits = _readout(h, params["Wk"], params["Wv"], params["seed"],
                             params["Wo"], params["Wn"], params["bn"])
    return (embed, logits)


# speedup vs baseline: 1.1290x; 1.1282x over previous
import functools

import jax
import jax.numpy as jnp
from jax import lax
from jax.experimental import pallas as pl
from jax.experimental.pallas import tpu as pltpu
from jax.experimental.pallas import tpu_sc as plsc

N = 10000
E = 320000
D_IN = 128
H = 256
HEADS = 4
DH = 64
NOTES = 128
L = 3

ROW_BLK = 1000


def _ln(z):
    m = jnp.mean(z, axis=-1, keepdims=True)
    v = jnp.mean((z - m) ** 2, axis=-1, keepdims=True)
    return (z - m) * lax.rsqrt(v + 1e-5)


def _mlp_block_kernel(hn_ref, agg_ref, res_ref, eps_ref, w0_ref, b0_ref,
                      w1_ref, b1_ref, w2_ref, b2_ref, h_out_ref, hn_out_ref,
                      *, has_residual):
    z = (1.0 + eps_ref[0, 0]) * hn_ref[...] + agg_ref[...]
    z = jnp.maximum(
        jnp.dot(z, w0_ref[...], preferred_element_type=jnp.float32)
        + b0_ref[...], 0.0)
    z = jnp.maximum(
        jnp.dot(z, w1_ref[...], preferred_element_type=jnp.float32)
        + b1_ref[...], 0.0)
    z = jnp.dot(z, w2_ref[...], preferred_element_type=jnp.float32) + b2_ref[...]
    z = _ln(z)
    if has_residual:
        z = z + res_ref[...]
    h_out_ref[...] = z
    hn_out_ref[...] = _ln(z)


def _mlp_block(hn, agg, res, eps, w0, b0, w1, b1, w2, b2):
    n, in_d = hn.shape
    has_residual = res is not None
    grid = (n // ROW_BLK,)
    in_specs = [
        pl.BlockSpec((ROW_BLK, in_d), lambda i: (i, 0)),
        pl.BlockSpec((ROW_BLK, in_d), lambda i: (i, 0)),
    ]
    args = [hn, agg]
    if has_residual:
        in_specs.append(pl.BlockSpec((ROW_BLK, H), lambda i: (i, 0)))
        args.append(res)
    else:
        in_specs.append(pl.BlockSpec(memory_space=pltpu.SMEM))
        args.append(jnp.zeros((1,), jnp.float32))
    in_specs.append(pl.BlockSpec(memory_space=pltpu.SMEM))
    args.append(eps.reshape(1, 1))
    for w, b in ((w0, b0), (w1, b1), (w2, b2)):
        in_specs.append(pl.BlockSpec((w.shape[0], H), lambda i: (0, 0)))
        in_specs.append(pl.BlockSpec((1, H), lambda i: (0, 0)))
        args.extend([w, b.reshape(1, H)])
    out_specs = [
        pl.BlockSpec((ROW_BLK, H), lambda i: (i, 0)),
        pl.BlockSpec((ROW_BLK, H), lambda i: (i, 0)),
    ]
    return pl.pallas_call(
        functools.partial(_mlp_block_kernel, has_residual=has_residual),
        grid=grid,
        in_specs=in_specs,
        out_specs=out_specs,
        out_shape=[
            jax.ShapeDtypeStruct((n, H), jnp.float32),
            jax.ShapeDtypeStruct((n, H), jnp.float32),
        ],
    )(*args)


def _ln_kernel(x_ref, o_ref):
    o_ref[...] = _ln(x_ref[...])


def _ln_call(x):
    n, d = x.shape
    return pl.pallas_call(
        _ln_kernel,
        grid=(n // ROW_BLK,),
        in_specs=[pl.BlockSpec((ROW_BLK, d), lambda i: (i, 0))],
        out_specs=pl.BlockSpec((ROW_BLK, d), lambda i: (i, 0)),
        out_shape=jax.ShapeDtypeStruct((n, d), jnp.float32),
    )(x)


NC, NS = 2, 16
CB = 1280
NCHUNK = E // CB
NBUF = 3
GSUB = tuple(range(0, CB, 128))
GLEN = (128,) * (CB // 128)


@functools.lru_cache(maxsize=None)
def _make_segsum(d):
    k_blocks = d // 8
    halves = (NC * NS) // k_blocks
    ne = E // halves
    nchunk_h = ne // CB
    nsuper = nchunk_h // NBUF
    shift = k_blocks.bit_length() - 1
    mesh = plsc.VectorSubcoreMesh(core_axis_name="c", subcore_axis_name="s")

    @functools.partial(
        pl.kernel,
        out_type=jax.ShapeDtypeStruct((halves * k_blocks, N, 8), jnp.float32),
        mesh=mesh,
        scratch_types=[
            [pltpu.VMEM((CB,), jnp.int32)] * NBUF,
            [pltpu.VMEM((CB,), jnp.int32)] * NBUF,
            [pltpu.VMEM((CB,), jnp.int32)] * NBUF,
            [pltpu.VMEM((CB, 8), jnp.float32)] * NBUF,
            pltpu.VMEM((N, 8), jnp.float32),
            [pltpu.SemaphoreType.DMA] * NBUF,
            [pltpu.SemaphoreType.DMA] * NBUF,
        ],
        compiler_params=pltpu.CompilerParams(use_tc_tiling_on_sc=False,
                                             needs_layout_passes=False),
    )
    def segsum(hn_hbm, src_hbm, dst_hbm, zeros_hbm, out_hbm,
               sbuf, dbuf, gidx, rows, acc, isem, gsem):
        c = lax.axis_index("c")
        s = lax.axis_index("s")
        w = s * NC + c

        pair_pat = jnp.where(lax.iota(jnp.int32, 16) < 8, 0, 1)
        col_pat = lax.iota(jnp.int32, 16) & 7
        m_lo = lax.iota(jnp.int32, 16) < 8
        m_hi = jnp.logical_not(m_lo)

        blk = w % k_blocks
        ebase = (w // k_blocks) * ne

        def fire_idx(i, b):
            off = ebase + i * CB
            pltpu.async_copy(src_hbm.at[pl.ds(off, CB)], sbuf[b], isem[b])
            pltpu.async_copy(dst_hbm.at[pl.ds(off, CB)], dbuf[b], isem[b])

        def wait_idx(b):
            pltpu.make_async_copy(src_hbm.at[pl.ds(0, CB)], sbuf[b],
                                  isem[b]).wait()
            pltpu.make_async_copy(dst_hbm.at[pl.ds(0, CB)], dbuf[b],
                                  isem[b]).wait()

        def compute_gidx(b, blk):
            def idx_body(q, carry):
                sv = sbuf[b][pl.ds(q * 16, 16)]
                gidx[b][pl.ds(q * 16, 16)] = (sv << shift) + blk
                return carry

            lax.fori_loop(0, CB // 16, idx_body, 0)

        def fire_gather(b):
            for o, n in zip(GSUB, GLEN):
                pltpu.async_copy(hn_hbm.at[gidx[b].at[pl.ds(o, n)]],
                                 rows[b].at[pl.ds(o, n)], gsem[b])

        def wait_gather(b):
            for o, n in zip(GSUB, GLEN):
                pltpu.make_async_copy(hn_hbm.at[gidx[b].at[pl.ds(o, n)]],
                                      rows[b].at[pl.ds(o, n)],
                                      gsem[b]).wait()

        def add_chunk(b):
            def add_body(g, ridx):
                for p in range(16):
                    dpair = plsc.load_gather(dbuf[b], [ridx])
                    v = plsc.load_gather(rows[b], [ridx, col_pat])
                    plsc.addupdate_scatter(acc, [dpair, col_pat], v,
                                           mask=m_lo)
                    plsc.addupdate_scatter(acc, [dpair, col_pat], v,
                                           mask=m_hi)
                    ridx = ridx + 2
                return ridx

            lax.fori_loop(0, CB // 32, add_body, pair_pat)

        pltpu.sync_copy(zeros_hbm, acc)

        fire_idx(0, 0)
        fire_idx(1, 1)
        wait_idx(0)
        compute_gidx(0, blk)
        fire_gather(0)

        def super_body(j, carry):
            for b in range(NBUF):
                i = j * NBUF + b
                bn = (b + 1) % NBUF
                bnn = (b + 2) % NBUF

                @pl.when(i + 1 < nchunk_h)
                def _():
                    wait_idx(bn)
                    compute_gidx(bn, blk)
                    fire_gather(bn)

                @pl.when(i + 2 < nchunk_h)
                def _():
                    fire_idx(i + 2, bnn)

                wait_gather(b)
                add_chunk(b)
            return carry

        lax.fori_loop(0, nsuper, super_body, 0)
        for i in range(nsuper * NBUF, nchunk_h):
            b = i % NBUF
            if i + 1 < nchunk_h:
                wait_idx((b + 1) % NBUF)
                compute_gidx((b + 1) % NBUF, blk)
                fire_gather((b + 1) % NBUF)
            if i + 2 < nchunk_h:
                fire_idx(i + 2, (b + 2) % NBUF)
            wait_gather(b)
            add_chunk(b)
        pltpu.sync_copy(acc, out_hbm.at[w])

    return segsum


def _readout_kernel(h_ref, wk_ref, wv_ref, s_ref, wo_ref, wn_ref, bn_ref,
                    embed_ref, logits_ref):
    h = h_ref[...]
    k = jnp.dot(h, wk_ref[...], preferred_element_type=jnp.float32)
    s = jnp.dot(k, s_ref[...], preferred_element_type=jnp.float32)
    m = jnp.max(s, axis=0, keepdims=True)
    w = jnp.exp(s - m)
    l = jnp.sum(w, axis=0, keepdims=True)
    attn = w / l
    v = jnp.dot(h, wv_ref[...], preferred_element_type=jnp.float32)
    wexp = jnp.repeat(attn, DH, axis=1)
    pooled = jnp.sum(wexp * v, axis=0, keepdims=True)
    embed = jnp.dot(pooled, wo_ref[...], preferred_element_type=jnp.float32)
    embed_ref[...] = embed
    logits_ref[...] = (
        jnp.dot(embed, wn_ref[...], preferred_element_type=jnp.float32)
        + bn_ref[...])


def _readout(h, wk, wv, seed, wo, wn, bn):
    smat = (jnp.eye(HEADS, dtype=jnp.float32)[:, None, :]
            * seed[:, :, None]).reshape(H, HEADS) / jnp.sqrt(float(DH))
    return pl.pallas_call(
        _readout_kernel,
        in_specs=[pl.BlockSpec((N, H), lambda: (0, 0)),
                  pl.BlockSpec((H, H), lambda: (0, 0)),
                  pl.BlockSpec((H, H), lambda: (0, 0)),
                  pl.BlockSpec((H, HEADS), lambda: (0, 0)),
                  pl.BlockSpec((H, H), lambda: (0, 0)),
                  pl.BlockSpec((H, NOTES), lambda: (0, 0)),
                  pl.BlockSpec((1, NOTES), lambda: (0, 0))],
        out_specs=[pl.BlockSpec((1, H), lambda: (0, 0)),
                   pl.BlockSpec((1, NOTES), lambda: (0, 0))],
        out_shape=[jax.ShapeDtypeStruct((1, H), jnp.float32),
                   jax.ShapeDtypeStruct((1, NOTES), jnp.float32)],
    )(h, wk, wv, smat, wo, wn, bn.reshape(1, NOTES))


def kernel(x, params, edge_index):
    src = edge_index[0]
    dst = edge_index[1]
    zeros_acc = jnp.zeros((N, 8), jnp.float32)

    hn = _ln_call(x)
    h = None
    residual = None
    for l in range(L):
        d = hn.shape[1]
        hn_r = hn.reshape(N * (d // 8), 8)
        agg_t = _make_segsum(d)(hn_r, src, dst, zeros_acc)
        kb = d // 8
        if agg_t.shape[0] != kb:
            agg_t = agg_t[:kb] + agg_t[kb:]
        agg = agg_t.transpose(1, 0, 2).reshape(N, d)
        h, hn = _mlp_block(
            hn, agg, residual, params["eps%d" % l],
            params["W%d_0" % l], params["b%d_0" % l],
            params["W%d_1" % l], params["b%d_1" % l],
            params["W%d_2" % l], params["b%d_2" % l])
        residual = h

    embed, logits = _readout(h, params["Wk"], params["Wv"], params["seed"],
                             params["Wo"], params["Wn"], params["bn"])
    return (embed, logits)


# speedup vs baseline: 1.1291x; 1.0000x over previous
import functools

import jax
import jax.numpy as jnp
from jax import lax
from jax.experimental import pallas as pl
from jax.experimental.pallas import tpu as pltpu
from jax.experimental.pallas import tpu_sc as plsc

N = 10000
E = 320000
D_IN = 128
H = 256
HEADS = 4
DH = 64
NOTES = 128
L = 3

ROW_BLK = 1000


def _ln(z):
    m = jnp.mean(z, axis=-1, keepdims=True)
    v = jnp.mean((z - m) ** 2, axis=-1, keepdims=True)
    return (z - m) * lax.rsqrt(v + 1e-5)


def _mlp_block_kernel(hn_ref, agg_ref, res_ref, eps_ref, w0_ref, b0_ref,
                      w1_ref, b1_ref, w2_ref, b2_ref, h_out_ref, hn_out_ref,
                      *, has_residual):
    def dotb(a, b):
        return jnp.dot(a.astype(jnp.bfloat16), b.astype(jnp.bfloat16),
                       preferred_element_type=jnp.float32)

    z = (1.0 + eps_ref[0, 0]) * hn_ref[...] + agg_ref[...]
    z = jnp.maximum(dotb(z, w0_ref[...]) + b0_ref[...], 0.0)
    z = jnp.maximum(dotb(z, w1_ref[...]) + b1_ref[...], 0.0)
    z = dotb(z, w2_ref[...]) + b2_ref[...]
    z = _ln(z)
    if has_residual:
        z = z + res_ref[...]
    h_out_ref[...] = z
    hn_out_ref[...] = _ln(z)


def _mlp_block(hn, agg, res, eps, w0, b0, w1, b1, w2, b2):
    n, in_d = hn.shape
    has_residual = res is not None
    grid = (n // ROW_BLK,)
    in_specs = [
        pl.BlockSpec((ROW_BLK, in_d), lambda i: (i, 0)),
        pl.BlockSpec((ROW_BLK, in_d), lambda i: (i, 0)),
    ]
    args = [hn, agg]
    if has_residual:
        in_specs.append(pl.BlockSpec((ROW_BLK, H), lambda i: (i, 0)))
        args.append(res)
    else:
        in_specs.append(pl.BlockSpec(memory_space=pltpu.SMEM))
        args.append(jnp.zeros((1,), jnp.float32))
    in_specs.append(pl.BlockSpec(memory_space=pltpu.SMEM))
    args.append(eps.reshape(1, 1))
    for w, b in ((w0, b0), (w1, b1), (w2, b2)):
        in_specs.append(pl.BlockSpec((w.shape[0], H), lambda i: (0, 0)))
        in_specs.append(pl.BlockSpec((1, H), lambda i: (0, 0)))
        args.extend([w, b.reshape(1, H)])
    out_specs = [
        pl.BlockSpec((ROW_BLK, H), lambda i: (i, 0)),
        pl.BlockSpec((ROW_BLK, H), lambda i: (i, 0)),
    ]
    return pl.pallas_call(
        functools.partial(_mlp_block_kernel, has_residual=has_residual),
        grid=grid,
        in_specs=in_specs,
        out_specs=out_specs,
        out_shape=[
            jax.ShapeDtypeStruct((n, H), jnp.float32),
            jax.ShapeDtypeStruct((n, H), jnp.float32),
        ],
    )(*args)


def _ln_kernel(x_ref, o_ref):
    o_ref[...] = _ln(x_ref[...])


def _ln_call(x):
    n, d = x.shape
    return pl.pallas_call(
        _ln_kernel,
        grid=(n // ROW_BLK,),
        in_specs=[pl.BlockSpec((ROW_BLK, d), lambda i: (i, 0))],
        out_specs=pl.BlockSpec((ROW_BLK, d), lambda i: (i, 0)),
        out_shape=jax.ShapeDtypeStruct((n, d), jnp.float32),
    )(x)


NC, NS = 2, 16
CB = 1280
NCHUNK = E // CB
NBUF = 3
GSUB = tuple(range(0, CB, 128))
GLEN = (128,) * (CB // 128)


@functools.lru_cache(maxsize=None)
def _make_segsum(d):
    k_blocks = d // 8
    halves = (NC * NS) // k_blocks
    ne = E // halves
    nchunk_h = ne // CB
    nsuper = nchunk_h // NBUF
    shift = k_blocks.bit_length() - 1
    mesh = plsc.VectorSubcoreMesh(core_axis_name="c", subcore_axis_name="s")

    @functools.partial(
        pl.kernel,
        out_type=jax.ShapeDtypeStruct((halves * k_blocks, N, 8), jnp.float32),
        mesh=mesh,
        scratch_types=[
            [pltpu.VMEM((CB,), jnp.int32)] * NBUF,
            [pltpu.VMEM((CB,), jnp.int32)] * NBUF,
            [pltpu.VMEM((CB,), jnp.int32)] * NBUF,
            [pltpu.VMEM((CB, 8), jnp.float32)] * NBUF,
            pltpu.VMEM((N, 8), jnp.float32),
            [pltpu.SemaphoreType.DMA] * NBUF,
            [pltpu.SemaphoreType.DMA] * NBUF,
        ],
        compiler_params=pltpu.CompilerParams(use_tc_tiling_on_sc=False,
                                             needs_layout_passes=False),
    )
    def segsum(hn_hbm, src_hbm, dst_hbm, zeros_hbm, out_hbm,
               sbuf, dbuf, gidx, rows, acc, isem, gsem):
        c = lax.axis_index("c")
        s = lax.axis_index("s")
        w = s * NC + c

        pair_pat = jnp.where(lax.iota(jnp.int32, 16) < 8, 0, 1)
        col_pat = lax.iota(jnp.int32, 16) & 7
        m_lo = lax.iota(jnp.int32, 16) < 8
        m_hi = jnp.logical_not(m_lo)

        blk = w % k_blocks
        ebase = (w // k_blocks) * ne

        def fire_idx(i, b):
            off = ebase + i * CB
            pltpu.async_copy(src_hbm.at[pl.ds(off, CB)], sbuf[b], isem[b])
            pltpu.async_copy(dst_hbm.at[pl.ds(off, CB)], dbuf[b], isem[b])

        def wait_idx(b):
            pltpu.make_async_copy(src_hbm.at[pl.ds(0, CB)], sbuf[b],
                                  isem[b]).wait()
            pltpu.make_async_copy(dst_hbm.at[pl.ds(0, CB)], dbuf[b],
                                  isem[b]).wait()

        def compute_gidx(b, blk):
            def idx_body(q, carry):
                sv = sbuf[b][pl.ds(q * 16, 16)]
                gidx[b][pl.ds(q * 16, 16)] = (sv << shift) + blk
                return carry

            lax.fori_loop(0, CB // 16, idx_body, 0)

        def fire_gather(b):
            for o, n in zip(GSUB, GLEN):
                pltpu.async_copy(hn_hbm.at[gidx[b].at[pl.ds(o, n)]],
                                 rows[b].at[pl.ds(o, n)], gsem[b])

        def wait_gather(b):
            for o, n in zip(GSUB, GLEN):
                pltpu.make_async_copy(hn_hbm.at[gidx[b].at[pl.ds(o, n)]],
                                      rows[b].at[pl.ds(o, n)],
                                      gsem[b]).wait()

        def add_chunk(b):
            def add_body(g, ridx):
                for p in range(16):
                    dpair = plsc.load_gather(dbuf[b], [ridx])
                    v = plsc.load_gather(rows[b], [ridx, col_pat])
                    plsc.addupdate_scatter(acc, [dpair, col_pat], v,
                                           mask=m_lo)
                    plsc.addupdate_scatter(acc, [dpair, col_pat], v,
                                           mask=m_hi)
                    ridx = ridx + 2
                return ridx

            lax.fori_loop(0, CB // 32, add_body, pair_pat)

        pltpu.sync_copy(zeros_hbm, acc)

        fire_idx(0, 0)
        fire_idx(1, 1)
        wait_idx(0)
        compute_gidx(0, blk)
        fire_gather(0)

        def super_body(j, carry):
            for b in range(NBUF):
                i = j * NBUF + b
                bn = (b + 1) % NBUF
                bnn = (b + 2) % NBUF

                @pl.when(i + 1 < nchunk_h)
                def _():
                    wait_idx(bn)
                    compute_gidx(bn, blk)
                    fire_gather(bn)

                @pl.when(i + 2 < nchunk_h)
                def _():
                    fire_idx(i + 2, bnn)

                wait_gather(b)
                add_chunk(b)
            return carry

        lax.fori_loop(0, nsuper, super_body, 0)
        for i in range(nsuper * NBUF, nchunk_h):
            b = i % NBUF
            if i + 1 < nchunk_h:
                wait_idx((b + 1) % NBUF)
                compute_gidx((b + 1) % NBUF, blk)
                fire_gather((b + 1) % NBUF)
            if i + 2 < nchunk_h:
                fire_idx(i + 2, (b + 2) % NBUF)
            wait_gather(b)
            add_chunk(b)
        pltpu.sync_copy(acc, out_hbm.at[w])

    return segsum


def _readout_kernel(h_ref, wk_ref, wv_ref, s_ref, wo_ref, wn_ref, bn_ref,
                    embed_ref, logits_ref):
    h = h_ref[...]
    k = jnp.dot(h, wk_ref[...], preferred_element_type=jnp.float32)
    s = jnp.dot(k, s_ref[...], preferred_element_type=jnp.float32)
    m = jnp.max(s, axis=0, keepdims=True)
    w = jnp.exp(s - m)
    l = jnp.sum(w, axis=0, keepdims=True)
    attn = w / l
    v = jnp.dot(h, wv_ref[...], preferred_element_type=jnp.float32)
    wexp = jnp.repeat(attn, DH, axis=1)
    pooled = jnp.sum(wexp * v, axis=0, keepdims=True)
    embed = jnp.dot(pooled, wo_ref[...], preferred_element_type=jnp.float32)
    embed_ref[...] = embed
    logits_ref[...] = (
        jnp.dot(embed, wn_ref[...], preferred_element_type=jnp.float32)
        + bn_ref[...])


def _readout(h, wk, wv, seed, wo, wn, bn):
    smat = (jnp.eye(HEADS, dtype=jnp.float32)[:, None, :]
            * seed[:, :, None]).reshape(H, HEADS) / jnp.sqrt(float(DH))
    return pl.pallas_call(
        _readout_kernel,
        in_specs=[pl.BlockSpec((N, H), lambda: (0, 0)),
                  pl.BlockSpec((H, H), lambda: (0, 0)),
                  pl.BlockSpec((H, H), lambda: (0, 0)),
                  pl.BlockSpec((H, HEADS), lambda: (0, 0)),
                  pl.BlockSpec((H, H), lambda: (0, 0)),
                  pl.BlockSpec((H, NOTES), lambda: (0, 0)),
                  pl.BlockSpec((1, NOTES), lambda: (0, 0))],
        out_specs=[pl.BlockSpec((1, H), lambda: (0, 0)),
                   pl.BlockSpec((1, NOTES), lambda: (0, 0))],
        out_shape=[jax.ShapeDtypeStruct((1, H), jnp.float32),
                   jax.ShapeDtypeStruct((1, NOTES), jnp.float32)],
    )(h, wk, wv, smat, wo, wn, bn.reshape(1, NOTES))


def kernel(x, params, edge_index):
    src = edge_index[0]
    dst = edge_index[1]
    zeros_acc = jnp.zeros((N, 8), jnp.float32)

    hn = _ln_call(x)
    h = None
    residual = None
    for l in range(L):
        d = hn.shape[1]
        hn_r = hn.reshape(N * (d // 8), 8)
        agg_t = _make_segsum(d)(hn_r, src, dst, zeros_acc)
        kb = d // 8
        if agg_t.shape[0] != kb:
            agg_t = agg_t[:kb] + agg_t[kb:]
        agg = agg_t.transpose(1, 0, 2).reshape(N, d)
        h, hn = _mlp_block(
            hn, agg, residual, params["eps%d" % l],
            params["W%d_0" % l], params["b%d_0" % l],
            params["W%d_1" % l], params["b%d_1" % l],
            params["W%d_2" % l], params["b%d_2" % l])
        residual = h

    embed, logits = _readout(h, params["Wk"], params["Wv"], params["seed"],
                             params["Wo"], params["Wn"], params["bn"])
    return (embed, logits)


# speedup vs baseline: 1.1291x; 1.0000x over previous
import functools

import jax
import jax.numpy as jnp
from jax import lax
from jax.experimental import pallas as pl
from jax.experimental.pallas import tpu as pltpu
from jax.experimental.pallas import tpu_sc as plsc

N = 10000
E = 320000
D_IN = 128
H = 256
HEADS = 4
DH = 64
NOTES = 128
L = 3

ROW_BLK = 1000


def _ln(z):
    m = jnp.mean(z, axis=-1, keepdims=True)
    v = jnp.mean((z - m) ** 2, axis=-1, keepdims=True)
    return (z - m) * lax.rsqrt(v + 1e-5)


def _mlp_block_kernel(hn_ref, agg_ref, res_ref, eps_ref, w0_ref, b0_ref,
                      w1_ref, b1_ref, w2_ref, b2_ref, h_out_ref, hn_out_ref,
                      *, has_residual):
    z = (1.0 + eps_ref[0, 0]) * hn_ref[...] + agg_ref[...]
    z = jnp.maximum(
        jnp.dot(z, w0_ref[...], preferred_element_type=jnp.float32)
        + b0_ref[...], 0.0)
    z = jnp.maximum(
        jnp.dot(z, w1_ref[...], preferred_element_type=jnp.float32)
        + b1_ref[...], 0.0)
    z = jnp.dot(z, w2_ref[...], preferred_element_type=jnp.float32) + b2_ref[...]
    z = _ln(z)
    if has_residual:
        z = z + res_ref[...]
    h_out_ref[...] = z
    hn_out_ref[...] = _ln(z)


def _mlp_block(hn, agg, res, eps, w0, b0, w1, b1, w2, b2):
    n, in_d = hn.shape
    has_residual = res is not None
    grid = (n // ROW_BLK,)
    in_specs = [
        pl.BlockSpec((ROW_BLK, in_d), lambda i: (i, 0)),
        pl.BlockSpec((ROW_BLK, in_d), lambda i: (i, 0)),
    ]
    args = [hn, agg]
    if has_residual:
        in_specs.append(pl.BlockSpec((ROW_BLK, H), lambda i: (i, 0)))
        args.append(res)
    else:
        in_specs.append(pl.BlockSpec(memory_space=pltpu.SMEM))
        args.append(jnp.zeros((1,), jnp.float32))
    in_specs.append(pl.BlockSpec(memory_space=pltpu.SMEM))
    args.append(eps.reshape(1, 1))
    for w, b in ((w0, b0), (w1, b1), (w2, b2)):
        in_specs.append(pl.BlockSpec((w.shape[0], H), lambda i: (0, 0)))
        in_specs.append(pl.BlockSpec((1, H), lambda i: (0, 0)))
        args.extend([w, b.reshape(1, H)])
    out_specs = [
        pl.BlockSpec((ROW_BLK, H), lambda i: (i, 0)),
        pl.BlockSpec((ROW_BLK, H), lambda i: (i, 0)),
    ]
    return pl.pallas_call(
        functools.partial(_mlp_block_kernel, has_residual=has_residual),
        grid=grid,
        in_specs=in_specs,
        out_specs=out_specs,
        out_shape=[
            jax.ShapeDtypeStruct((n, H), jnp.float32),
            jax.ShapeDtypeStruct((n, H), jnp.float32),
        ],
    )(*args)


def _ln_kernel(x_ref, o_ref):
    o_ref[...] = _ln(x_ref[...])


def _ln_call(x):
    n, d = x.shape
    return pl.pallas_call(
        _ln_kernel,
        grid=(n // ROW_BLK,),
        in_specs=[pl.BlockSpec((ROW_BLK, d), lambda i: (i, 0))],
        out_specs=pl.BlockSpec((ROW_BLK, d), lambda i: (i, 0)),
        out_shape=jax.ShapeDtypeStruct((n, d), jnp.float32),
    )(x)


NC, NS = 2, 16
CB = 1280
NCHUNK = E // CB
NBUF = 3
GSUB = tuple(range(0, CB, 128))
GLEN = (128,) * (CB // 128)


@functools.lru_cache(maxsize=None)
def _make_segsum(d):
    k_blocks = d // 8
    halves = (NC * NS) // k_blocks
    ne = E // halves
    nchunk_h = ne // CB
    nsuper = nchunk_h // NBUF
    shift = k_blocks.bit_length() - 1
    mesh = plsc.VectorSubcoreMesh(core_axis_name="c", subcore_axis_name="s")

    @functools.partial(
        pl.kernel,
        out_type=jax.ShapeDtypeStruct((halves * k_blocks, N, 8), jnp.float32),
        mesh=mesh,
        scratch_types=[
            [pltpu.VMEM((CB,), jnp.int32)] * NBUF,
            [pltpu.VMEM((CB,), jnp.int32)] * NBUF,
            [pltpu.VMEM((CB,), jnp.int32)] * NBUF,
            [pltpu.VMEM((CB, 8), jnp.float32)] * NBUF,
            pltpu.VMEM((N, 8), jnp.float32),
            [pltpu.SemaphoreType.DMA] * NBUF,
            [pltpu.SemaphoreType.DMA] * NBUF,
        ],
        compiler_params=pltpu.CompilerParams(use_tc_tiling_on_sc=False,
                                             needs_layout_passes=False),
    )
    def segsum(hn_hbm, src_hbm, dst_hbm, zeros_hbm, out_hbm,
               sbuf, dbuf, gidx, rows, acc, isem, gsem):
        c = lax.axis_index("c")
        s = lax.axis_index("s")
        w = s * NC + c

        pair_pat = jnp.where(lax.iota(jnp.int32, 16) < 8, 0, 1)
        col_pat = lax.iota(jnp.int32, 16) & 7
        m_lo = lax.iota(jnp.int32, 16) < 8
        m_hi = jnp.logical_not(m_lo)

        blk = w % k_blocks
        ebase = (w // k_blocks) * ne

        def fire_idx(i, b):
            off = ebase + i * CB
            pltpu.async_copy(src_hbm.at[pl.ds(off, CB)], sbuf[b], isem[b])
            pltpu.async_copy(dst_hbm.at[pl.ds(off, CB)], dbuf[b], isem[b])

        def wait_idx(b):
            pltpu.make_async_copy(src_hbm.at[pl.ds(0, CB)], sbuf[b],
                                  isem[b]).wait()
            pltpu.make_async_copy(dst_hbm.at[pl.ds(0, CB)], dbuf[b],
                                  isem[b]).wait()

        def compute_gidx(b, blk):
            def idx_body(q, carry):
                sv = sbuf[b][pl.ds(q * 16, 16)]
                gidx[b][pl.ds(q * 16, 16)] = (sv << shift) + blk
                return carry

            lax.fori_loop(0, CB // 16, idx_body, 0)

        def fire_gather(b):
            for o, n in zip(GSUB, GLEN):
                pltpu.async_copy(hn_hbm.at[gidx[b].at[pl.ds(o, n)]],
                                 rows[b].at[pl.ds(o, n)], gsem[b])

        def wait_gather(b):
            for o, n in zip(GSUB, GLEN):
                pltpu.make_async_copy(hn_hbm.at[gidx[b].at[pl.ds(o, n)]],
                                      rows[b].at[pl.ds(o, n)],
                                      gsem[b]).wait()

        def add_chunk(b):
            def add_body(g, ridx):
                for p in range(16):
                    dpair = plsc.load_gather(dbuf[b], [ridx])
                    v = plsc.load_gather(rows[b], [ridx, col_pat])
                    plsc.addupdate_scatter(acc, [dpair, col_pat], v,
                                           mask=m_lo)
                    plsc.addupdate_scatter(acc, [dpair, col_pat], v,
                                           mask=m_hi)
                    ridx = ridx + 2
                return ridx

            lax.fori_loop(0, CB // 32, add_body, pair_pat)

        pltpu.sync_copy(zeros_hbm, acc)

        fire_idx(0, 0)
        fire_idx(1, 1)
        wait_idx(0)
        compute_gidx(0, blk)
        fire_gather(0)

        def super_body(j, carry):
            for b in range(NBUF):
                i = j * NBUF + b
                bn = (b + 1) % NBUF
                bnn = (b + 2) % NBUF

                @pl.when(i + 1 < nchunk_h)
                def _():
                    wait_idx(bn)
                    compute_gidx(bn, blk)
                    fire_gather(bn)

                @pl.when(i + 2 < nchunk_h)
                def _():
                    fire_idx(i + 2, bnn)

                wait_gather(b)
                add_chunk(b)
            return carry

        lax.fori_loop(0, nsuper, super_body, 0)
        for i in range(nsuper * NBUF, nchunk_h):
            b = i % NBUF
            if i + 1 < nchunk_h:
                wait_idx((b + 1) % NBUF)
                compute_gidx((b + 1) % NBUF, blk)
                fire_gather((b + 1) % NBUF)
            if i + 2 < nchunk_h:
                fire_idx(i + 2, (b + 2) % NBUF)
            wait_gather(b)
            add_chunk(b)
        pltpu.sync_copy(acc, out_hbm.at[w])

    return segsum


def _readout_kernel(h_ref, wk_ref, wv_ref, s_ref, wo_ref, wn_ref, bn_ref,
                    embed_ref, logits_ref):
    h = h_ref[...]
    k = jnp.dot(h, wk_ref[...], preferred_element_type=jnp.float32)
    s = jnp.dot(k, s_ref[...], preferred_element_type=jnp.float32)
    m = jnp.max(s, axis=0, keepdims=True)
    w = jnp.exp(s - m)
    l = jnp.sum(w, axis=0, keepdims=True)
    attn = w / l
    v = jnp.dot(h, wv_ref[...], preferred_element_type=jnp.float32)
    wexp = jnp.repeat(attn, DH, axis=1)
    pooled = jnp.sum(wexp * v, axis=0, keepdims=True)
    embed = jnp.dot(pooled, wo_ref[...], preferred_element_type=jnp.float32)
    embed_ref[...] = embed
    logits_ref[...] = (
        jnp.dot(embed, wn_ref[...], preferred_element_type=jnp.float32)
        + bn_ref[...])


def _readout(h, wk, wv, seed, wo, wn, bn):
    smat = (jnp.eye(HEADS, dtype=jnp.float32)[:, None, :]
            * seed[:, :, None]).reshape(H, HEADS) / jnp.sqrt(float(DH))
    return pl.pallas_call(
        _readout_kernel,
        in_specs=[pl.BlockSpec((N, H), lambda: (0, 0)),
                  pl.BlockSpec((H, H), lambda: (0, 0)),
                  pl.BlockSpec((H, H), lambda: (0, 0)),
                  pl.BlockSpec((H, HEADS), lambda: (0, 0)),
                  pl.BlockSpec((H, H), lambda: (0, 0)),
                  pl.BlockSpec((H, NOTES), lambda: (0, 0)),
                  pl.BlockSpec((1, NOTES), lambda: (0, 0))],
        out_specs=[pl.BlockSpec((1, H), lambda: (0, 0)),
                   pl.BlockSpec((1, NOTES), lambda: (0, 0))],
        out_shape=[jax.ShapeDtypeStruct((1, H), jnp.float32),
                   jax.ShapeDtypeStruct((1, NOTES), jnp.float32)],
    )(h, wk, wv, smat, wo, wn, bn.reshape(1, NOTES))


def kernel(x, params, edge_index):
    src = edge_index[0]
    dst = edge_index[1]
    zeros_acc = jnp.zeros((N, 8), jnp.float32)

    hn = _ln_call(x)
    h = None
    residual = None
    for l in range(L):
        d = hn.shape[1]
        hn_r = hn.reshape(N * (d // 8), 8)
        agg_t = _make_segsum(d)(hn_r, src, dst, zeros_acc)
        kb = d // 8
        if agg_t.shape[0] != kb:
            agg_t = agg_t[:kb] + agg_t[kb:]
        agg = agg_t.transpose(1, 0, 2).reshape(N, d)
        h, hn = _mlp_block(
            hn, agg, residual, params["eps%d" % l],
            params["W%d_0" % l], params["b%d_0" % l],
            params["W%d_1" % l], params["b%d_1" % l],
            params["W%d_2" % l], params["b%d_2" % l])
        residual = h

    embed, logits = _readout(h, params["Wk"], params["Wv"], params["seed"],
                             params["Wo"], params["Wn"], params["bn"])
    return (embed, logits)
